# baseline (speedup 1.0000x reference)
import functools

import numpy as np
import jax
import jax.numpy as jnp
from jax import lax
from jax.experimental import pallas as pl
from jax.experimental.pallas import tpu as pltpu
from jax.experimental.pallas import tpu_sc as plsc

D_MODEL = 64
LANES = 16
RING = 4
NC = 2
NS = 16
NW = NC * NS


def _make_sc_kernel(B, T):
    n_bblk = B // 128
    bblk_per_worker = n_bblk // NW
    n_tblk = T // 8
    scale = jnp.float32(8.0)

    mesh = plsc.VectorSubcoreMesh(core_axis_name="c", subcore_axis_name="s")

    @functools.partial(
        pl.kernel,
        out_type=jax.ShapeDtypeStruct((T, 8, n_bblk, 8 * 128), jnp.float32),
        mesh=mesh,
        compiler_params=pltpu.CompilerParams(
            use_tc_tiling_on_sc=False, needs_layout_passes=False),
        scratch_types=[
            pltpu.VMEM((n_tblk, 8, 128), jnp.int32),
            pltpu.VMEM((RING, 128, D_MODEL), jnp.float32),
            pltpu.VMEM((RING, 8, 8 * 128), jnp.float32),
            pltpu.VMEM((T, D_MODEL), jnp.float32),
        ] + [pltpu.SemaphoreType.DMA] * (2 * RING),
    )
    def sc_kernel(x_hbm, tok_hbm, pos_hbm, out_hbm, idx_v, rows_v, o_v, pos_v,
                  *sems):
        gsem = sems[0:RING]
        osem = sems[RING:2 * RING]
        wid = lax.axis_index("s") * NC + lax.axis_index("c")

        lane = lax.iota(jnp.int32, LANES)
        dblk_c = [(lane + j * LANES) >> 3 for j in range(4)]
        inner_c = [((lane + j * LANES) & 7) * 128 for j in range(4)]

        pltpu.sync_copy(pos_hbm.at[pl.ds(0, T)], pos_v)

        def gather_cp(u, r):
            return pltpu.make_async_copy(
                tok_hbm.at[idx_v.at[lax.div(u, 8), lax.rem(u, 8)]],
                rows_v.at[r], gsem[r])

        for wb in range(bblk_per_worker):
            bblk = wid * bblk_per_worker + wb
            pltpu.sync_copy(x_hbm.at[:, bblk], idx_v)

            def out_cp(t, r):
                return pltpu.make_async_copy(
                    o_v.at[r], out_hbm.at[t, :, bblk], osem[r])

            for u in range(RING - 1):
                gather_cp(u, u % RING).start()

            def unit_body(t4, carry):
                for rs in range(RING):
                    t = t4 * RING + rs
                    gather_cp(t, rs).wait()

                    ahead_ok = (t4 < T // RING - 1) if rs > 0 else True

                    def _fire():
                        gather_cp(t + (RING - 1),
                                  (rs + (RING - 1)) % RING).start()
                    if ahead_ok is True:
                        _fire()
                    else:
                        pl.when(t4 < T // RING - 1)(_fire)

                    @pl.when(t4 >= 1)
                    def _():
                        out_cp(t - RING, rs).wait()

                    pos_j = [pos_v[t, pl.ds(j * LANES, LANES)]
                             for j in range(4)]

                    @plsc.parallel_loop(0, 128, unroll=2)
                    def _(rr):
                        rvec = jnp.full((LANES,), rr, jnp.int32)
                        for j in range(4):
                            vec = (rows_v[rs, rr, pl.ds(j * LANES, LANES)]
                                   * scale + pos_j[j])
                            plsc.store_scatter(
                                o_v.at[rs],
                                [dblk_c[j], inner_c[j] + rvec], vec)

                    out_cp(t, rs).start()
                return carry

            lax.fori_loop(0, T // RING, unit_body, 0)

            for k in range(RING):
                t = T - RING + k
                out_cp(t, t % RING).wait()

    return sc_kernel


def kernel(x, token_table, pos_table):
    B, T = x.shape
    assert B % (128 * NW) == 0 and T % 8 == 0
    x5 = x.astype(jnp.int32).reshape(B // 128, 128, T // 8, 8)
    x5 = x5.transpose((2, 0, 3, 1))
    sc_kernel = _make_sc_kernel(B, T)
    o5 = sc_kernel(x5, token_table, pos_table)
    o5 = o5.reshape(T, 8, B // 128, 8, 128)
    return o5.transpose((2, 4, 0, 1, 3)).reshape(B, T, D_MODEL)


# speedup vs baseline: 2.9467x; 2.9467x over previous
import functools

import numpy as np
import jax
import jax.numpy as jnp
from jax import lax
from jax.experimental import pallas as pl
from jax.experimental.pallas import tpu as pltpu
from jax.experimental.pallas import tpu_sc as plsc

D_MODEL = 64
LANES = 16
RING = 4
NC = 2
NS = 16
NW = NC * NS


def _make_sc_kernel(B, T):
    n_bblk = B // 128
    bblk_per_worker = n_bblk // NW
    n_tblk = T // 8
    scale = jnp.float32(8.0)

    mesh = plsc.VectorSubcoreMesh(core_axis_name="c", subcore_axis_name="s")

    @functools.partial(
        pl.kernel,
        out_type=jax.ShapeDtypeStruct((T, 8, n_bblk, 8, 128), jnp.float32),
        mesh=mesh,
        compiler_params=pltpu.CompilerParams(
            use_tc_tiling_on_sc=False, needs_layout_passes=False),
        scratch_types=[
            pltpu.VMEM((n_tblk, 8, 128), jnp.int32),
            pltpu.VMEM((RING, 128, D_MODEL), jnp.float32),
            pltpu.VMEM((RING, 8, 8, 129), jnp.float32),
            pltpu.VMEM((T, D_MODEL), jnp.float32),
        ] + [pltpu.SemaphoreType.DMA] * (2 * RING),
    )
    def sc_kernel(x_hbm, tok_hbm, pos_hbm, out_hbm, idx_v, rows_v, o_v, pos_v,
                  *sems):
        gsem = sems[0:RING]
        osem = sems[RING:2 * RING]
        wid = lax.axis_index("s") * NC + lax.axis_index("c")

        lane = lax.iota(jnp.int32, LANES)
        dblk_c = [(lane + j * LANES) >> 3 for j in range(4)]
        dsub_c = [(lane + j * LANES) & 7 for j in range(4)]

        pltpu.sync_copy(pos_hbm.at[pl.ds(0, T)], pos_v)

        def gather_cp(u, r):
            return pltpu.make_async_copy(
                tok_hbm.at[idx_v.at[lax.div(u, 8), lax.rem(u, 8)]],
                rows_v.at[r], gsem[r])

        for wb in range(bblk_per_worker):
            bblk = wid * bblk_per_worker + wb
            pltpu.sync_copy(x_hbm.at[:, bblk], idx_v)

            def out_cp(t, r):
                return pltpu.make_async_copy(
                    o_v.at[r, :, :, pl.ds(0, 128)],
                    out_hbm.at[t, :, bblk], osem[r])

            for u in range(RING - 1):
                gather_cp(u, u % RING).start()

            def unit_body(t4, carry):
                for rs in range(RING):
                    t = t4 * RING + rs
                    gather_cp(t, rs).wait()

                    ahead_ok = (t4 < T // RING - 1) if rs > 0 else True

                    def _fire():
                        gather_cp(t + (RING - 1),
                                  (rs + (RING - 1)) % RING).start()
                    if ahead_ok is True:
                        _fire()
                    else:
                        pl.when(t4 < T // RING - 1)(_fire)

                    @pl.when(t4 >= 1)
                    def _():
                        out_cp(t - RING, rs).wait()

                    pos_j = [pos_v[t, pl.ds(j * LANES, LANES)]
                             for j in range(4)]

                    @plsc.parallel_loop(0, 128, unroll=2)
                    def _(rr):
                        rvec = jnp.full((LANES,), rr, jnp.int32)
                        for j in range(4):
                            vec = (rows_v[rs, rr, pl.ds(j * LANES, LANES)]
                                   * scale + pos_j[j])
                            plsc.store_scatter(
                                o_v.at[rs],
                                [dblk_c[j], dsub_c[j], rvec], vec)

                    out_cp(t, rs).start()
                return carry

            lax.fori_loop(0, T // RING, unit_body, 0)

            for k in range(RING):
                t = T - RING + k
                out_cp(t, t % RING).wait()

    return sc_kernel


def kernel(x, token_table, pos_table):
    B, T = x.shape
    assert B % (128 * NW) == 0 and T % 8 == 0
    x5 = x.astype(jnp.int32).reshape(B // 128, 128, T // 8, 8)
    x5 = x5.transpose((2, 0, 3, 1))
    sc_kernel = _make_sc_kernel(B, T)
    o5 = sc_kernel(x5, token_table, pos_table)
    return o5.transpose((2, 4, 0, 1, 3)).reshape(B, T, D_MODEL)


# speedup vs baseline: 2.9493x; 1.0009x over previous
import functools

import numpy as np
import jax
import jax.numpy as jnp
from jax import lax
from jax.experimental import pallas as pl
from jax.experimental.pallas import tpu as pltpu
from jax.experimental.pallas import tpu_sc as plsc

D_MODEL = 64
LANES = 16
RING = 4
NC = 2
NS = 16
NW = NC * NS


def _make_sc_kernel(B, T):
    n_bblk = B // 128
    bblk_per_worker = n_bblk // NW
    n_tblk = T // 8
    scale = jnp.float32(8.0)

    mesh = plsc.VectorSubcoreMesh(core_axis_name="c", subcore_axis_name="s")

    @functools.partial(
        pl.kernel,
        out_type=jax.ShapeDtypeStruct((T, 8, n_bblk, 8, 128), jnp.float32),
        mesh=mesh,
        compiler_params=pltpu.CompilerParams(
            use_tc_tiling_on_sc=False, needs_layout_passes=False),
        scratch_types=[
            pltpu.VMEM((n_tblk, 8, 128), jnp.int32),
            pltpu.VMEM((RING, 128, D_MODEL), jnp.float32),
            pltpu.VMEM((RING, 8, 8, 129), jnp.float32),
            pltpu.VMEM((T, D_MODEL), jnp.float32),
        ] + [pltpu.SemaphoreType.DMA] * (2 * RING),
    )
    def sc_kernel(x_hbm, tok_hbm, pos_hbm, out_hbm, idx_v, rows_v, o_v, pos_v,
                  *sems):
        gsem = sems[0:RING]
        osem = sems[RING:2 * RING]
        wid = lax.axis_index("s") * NC + lax.axis_index("c")

        lane = lax.iota(jnp.int32, LANES)
        dblk_c = [(lane + j * LANES) >> 3 for j in range(4)]
        dsub_c = [(lane + j * LANES) & 7 for j in range(4)]

        pltpu.sync_copy(pos_hbm.at[pl.ds(0, T)], pos_v)

        def gather_cp(u, r):
            return pltpu.make_async_copy(
                tok_hbm.at[idx_v.at[lax.div(u, 8), lax.rem(u, 8)]],
                rows_v.at[r], gsem[r])

        for wb in range(bblk_per_worker):
            bblk = wid * bblk_per_worker + wb
            pltpu.sync_copy(x_hbm.at[:, bblk], idx_v)

            def out_cp(t, r):
                return pltpu.make_async_copy(
                    o_v.at[r, :, :, pl.ds(0, 128)],
                    out_hbm.at[t, :, bblk], osem[r])

            for u in range(RING - 1):
                gather_cp(u, u % RING).start()

            def unit_body(t4, carry):
                for rs in range(RING):
                    t = t4 * RING + rs
                    gather_cp(t, rs).wait()

                    ahead_ok = (t4 < T // RING - 1) if rs > 0 else True

                    def _fire():
                        gather_cp(t + (RING - 1),
                                  (rs + (RING - 1)) % RING).start()
                    if ahead_ok is True:
                        _fire()
                    else:
                        pl.when(t4 < T // RING - 1)(_fire)

                    @pl.when(t4 >= 1)
                    def _():
                        out_cp(t - RING, rs).wait()

                    pos_j = [pos_v[t, pl.ds(j * LANES, LANES)]
                             for j in range(4)]

                    @plsc.parallel_loop(0, 128, unroll=4)
                    def _(rr):
                        rvec = jnp.full((LANES,), rr, jnp.int32)
                        for j in range(4):
                            vec = (rows_v[rs, rr, pl.ds(j * LANES, LANES)]
                                   * scale + pos_j[j])
                            plsc.store_scatter(
                                o_v.at[rs],
                                [dblk_c[j], dsub_c[j], rvec], vec)

                    out_cp(t, rs).start()
                return carry

            lax.fori_loop(0, T // RING, unit_body, 0)

            for k in range(RING):
                t = T - RING + k
                out_cp(t, t % RING).wait()

    return sc_kernel


def kernel(x, token_table, pos_table):
    B, T = x.shape
    assert B % (128 * NW) == 0 and T % 8 == 0
    x5 = x.astype(jnp.int32).reshape(B // 128, 128, T // 8, 8)
    x5 = x5.transpose((2, 0, 3, 1))
    sc_kernel = _make_sc_kernel(B, T)
    o5 = sc_kernel(x5, token_table, pos_table)
    return o5.transpose((2, 4, 0, 1, 3)).reshape(B, T, D_MODEL)


# speedup vs baseline: 2.9665x; 1.0058x over previous
import functools

import numpy as np
import jax
import jax.numpy as jnp
from jax import lax
from jax.experimental import pallas as pl
from jax.experimental.pallas import tpu as pltpu
from jax.experimental.pallas import tpu_sc as plsc

D_MODEL = 64
LANES = 16
RING = 5
NC = 2
NS = 16
NW = NC * NS


def _make_sc_kernel(B, T):
    n_bblk = B // 128
    bblk_per_worker = n_bblk // NW
    n_tblk = T // 8
    scale = jnp.float32(8.0)

    mesh = plsc.VectorSubcoreMesh(core_axis_name="c", subcore_axis_name="s")

    @functools.partial(
        pl.kernel,
        out_type=jax.ShapeDtypeStruct((T, 8, n_bblk, 8, 128), jnp.float32),
        mesh=mesh,
        compiler_params=pltpu.CompilerParams(
            use_tc_tiling_on_sc=False, needs_layout_passes=False),
        scratch_types=[
            pltpu.VMEM((n_tblk, 8, 128), jnp.int32),
            pltpu.VMEM((RING, 128, D_MODEL), jnp.float32),
            pltpu.VMEM((RING, 8, 8, 129), jnp.float32),
            pltpu.VMEM((T, D_MODEL), jnp.float32),
        ] + [pltpu.SemaphoreType.DMA] * (2 * RING),
    )
    def sc_kernel(x_hbm, tok_hbm, pos_hbm, out_hbm, idx_v, rows_v, o_v, pos_v,
                  *sems):
        gsem = sems[0:RING]
        osem = sems[RING:2 * RING]
        wid = lax.axis_index("s") * NC + lax.axis_index("c")

        lane = lax.iota(jnp.int32, LANES)
        dblk_c = [(lane + j * LANES) >> 3 for j in range(4)]
        dsub_c = [(lane + j * LANES) & 7 for j in range(4)]

        pltpu.sync_copy(pos_hbm.at[pl.ds(0, T)], pos_v)

        def gather_cp(u, r):
            return pltpu.make_async_copy(
                tok_hbm.at[idx_v.at[lax.div(u, 8), lax.rem(u, 8)]],
                rows_v.at[r], gsem[r])

        for wb in range(bblk_per_worker):
            bblk = wid * bblk_per_worker + wb
            pltpu.sync_copy(x_hbm.at[:, bblk], idx_v)

            def out_cp(t, r):
                return pltpu.make_async_copy(
                    o_v.at[r, :, :, pl.ds(0, 128)],
                    out_hbm.at[t, :, bblk], osem[r])

            for u in range(RING - 1):
                gather_cp(u, u % RING).start()

            def unit_body(t4, carry):
                for rs in range(RING):
                    t = t4 * RING + rs
                    gather_cp(t, rs).wait()

                    ahead_ok = (t4 < T // RING - 1) if rs > 0 else True

                    def _fire():
                        gather_cp(t + (RING - 1),
                                  (rs + (RING - 1)) % RING).start()
                    if ahead_ok is True:
                        _fire()
                    else:
                        pl.when(t4 < T // RING - 1)(_fire)

                    @pl.when(t4 >= 1)
                    def _():
                        out_cp(t - RING, rs).wait()

                    pos_j = [pos_v[t, pl.ds(j * LANES, LANES)]
                             for j in range(4)]

                    @plsc.parallel_loop(0, 128, unroll=4)
                    def _(rr):
                        rvec = jnp.full((LANES,), rr, jnp.int32)
                        for j in range(4):
                            vec = (rows_v[rs, rr, pl.ds(j * LANES, LANES)]
                                   * scale + pos_j[j])
                            plsc.store_scatter(
                                o_v.at[rs],
                                [dblk_c[j], dsub_c[j], rvec], vec)

                    out_cp(t, rs).start()
                return carry

            lax.fori_loop(0, T // RING, unit_body, 0)

            for k in range(RING):
                t = T - RING + k
                out_cp(t, t % RING).wait()

    return sc_kernel


def kernel(x, token_table, pos_table):
    B, T = x.shape
    assert B % (128 * NW) == 0 and T % 8 == 0
    x5 = x.astype(jnp.int32).reshape(B // 128, 128, T // 8, 8)
    x5 = x5.transpose((2, 0, 3, 1))
    sc_kernel = _make_sc_kernel(B, T)
    o5 = sc_kernel(x5, token_table, pos_table)
    return o5.transpose((2, 4, 0, 1, 3)).reshape(B, T, D_MODEL)
